# Initial kernel scaffold; baseline (speedup 1.0000x reference)
import jax
import jax.numpy as jnp
from jax.experimental import pallas as pl


def kernel(x, coords, emb_x, emb_y):
    raise NotImplementedError("write your pallas kernel here")


# speedup vs baseline: 1.6956x; 1.6956x over previous
import jax
import jax.numpy as jnp
from jax import lax
from jax.experimental import pallas as pl
from jax.experimental.pallas import tpu as pltpu
from jax.experimental.pallas import tpu_sc as plsc

_TILE = 16
_HALF = 128
_DIM = 256
_NC = 2
_NS = 16
_NW = _NC * _NS
_LANES = 16

_CHUNK = 64
_GROWS = 2 * _CHUNK


def _pe_body(x_hbm, c1_hbm, c2_hbm, emb_hbm, out_hbm, stage_hbm,
             cbuf, acc2, gbuf, c1b, c2b, idxbuf, xbuf, gatbuf, semg):
    seq = x_hbm.shape[0]
    rows_per_w = seq // _NW
    nchunk = rows_per_w // _CHUNK
    min_per_sub = seq // _NS

    s = lax.axis_index("s")
    c = lax.axis_index("c")
    wid = s * _NC + c
    wbase = wid * rows_per_w

    mbase = s * min_per_sub
    for slot, src in ((0, c1_hbm), (1, c2_hbm)):
        pltpu.sync_copy(src.at[pl.ds(mbase, min_per_sub)], cbuf)
        acc2[pl.ds(slot * _LANES, _LANES)] = cbuf[pl.ds(0, _LANES)]

        def _minstep(i, _, _slot=slot):
            acc2[pl.ds(_slot * _LANES, _LANES)] = jnp.minimum(
                acc2[pl.ds(_slot * _LANES, _LANES)],
                cbuf[pl.ds(i * _LANES, _LANES)])
            return 0

        lax.fori_loop(1, min_per_sub // _LANES, _minstep, 0)

    pltpu.sync_copy(acc2, stage_hbm.at[c * _NS + s])
    plsc.subcore_barrier()
    pltpu.sync_copy(stage_hbm.at[pl.ds(c * _NS, _NS)], gbuf)

    m1 = gbuf[0, pl.ds(0, _LANES)]
    m2 = gbuf[0, pl.ds(_LANES, _LANES)]
    for t in range(1, _NS):
        m1 = jnp.minimum(m1, gbuf[t, pl.ds(0, _LANES)])
        m2 = jnp.minimum(m2, gbuf[t, pl.ds(_LANES, _LANES)])
    mx = m1[0]
    my = m2[0]
    for t in range(1, _LANES):
        mx = jnp.minimum(mx, m1[t])
        my = jnp.minimum(my, m2[t])

    pltpu.sync_copy(c1_hbm.at[pl.ds(wbase, rows_per_w)], c1b)
    pltpu.sync_copy(c2_hbm.at[pl.ds(wbase, rows_per_w)], c2b)

    mxv = jnp.full((_LANES,), mx, jnp.int32)
    myv = jnp.full((_LANES,), my, jnp.int32)
    tilev = jnp.full((_LANES,), _TILE, jnp.int32)
    offv = jnp.full((_LANES,), emb_hbm.shape[0] // 2, jnp.int32)
    groups_per_chunk = _CHUNK // _LANES

    for ci in range(nchunk):
        row0 = wbase + ci * _CHUNK
        for k in range(groups_per_chunk):
            j = ci * groups_per_chunk + k
            vx = lax.div(c1b[pl.ds(j * _LANES, _LANES)] - mxv, tilev)
            idxbuf[pl.ds(k * _LANES, _LANES)] = vx
            vy = lax.div(c2b[pl.ds(j * _LANES, _LANES)] - myv, tilev) + offv
            idxbuf[pl.ds(_CHUNK + k * _LANES, _LANES)] = vy
        pltpu.sync_copy(x_hbm.at[pl.ds(row0, _CHUNK)], xbuf)
        pltpu.async_copy(emb_hbm.at[idxbuf], gatbuf, semg).wait()

        def _addrow(r, _):
            for h in range(_HALF // _LANES):
                sl = pl.ds(h * _LANES, _LANES)
                sr = pl.ds(_HALF + h * _LANES, _LANES)
                xbuf[r, sl] = xbuf[r, sl] + gatbuf[r, sl]
                xbuf[r, sr] = xbuf[r, sr] + gatbuf[_CHUNK + r, sl]
            return 0

        lax.fori_loop(0, _CHUNK, _addrow, 0)
        pltpu.sync_copy(xbuf, out_hbm.at[pl.ds(row0, _CHUNK)])


def kernel(x, coords, emb_x, emb_y):
    seq, dim = x.shape
    c1 = coords[:, 1].astype(jnp.int32)
    c2 = coords[:, 2].astype(jnp.int32)
    emb_cat = jnp.concatenate([emb_x, emb_y], axis=0)

    rows_per_w = seq // _NW
    min_per_sub = seq // _NS

    run = pl.kernel(
        _pe_body,
        out_type=(
            jax.ShapeDtypeStruct((seq, dim), jnp.float32),
            jax.ShapeDtypeStruct((_NW, 2 * _LANES), jnp.int32),
        ),
        mesh=plsc.VectorSubcoreMesh(core_axis_name="c", subcore_axis_name="s"),
        scratch_types=[
            pltpu.VMEM((min_per_sub,), jnp.int32),
            pltpu.VMEM((2 * _LANES,), jnp.int32),
            pltpu.VMEM((_NS, 2 * _LANES), jnp.int32),
            pltpu.VMEM((rows_per_w,), jnp.int32),
            pltpu.VMEM((rows_per_w,), jnp.int32),
            pltpu.VMEM((_GROWS,), jnp.int32),
            pltpu.VMEM((_CHUNK, _DIM), jnp.float32),
            pltpu.VMEM((_GROWS, _HALF), jnp.float32),
            pltpu.SemaphoreType.DMA,
        ],
    )
    out, _ = run(x, c1, c2, emb_cat)
    return out


# speedup vs baseline: 2.2980x; 1.3553x over previous
import jax
import jax.numpy as jnp
from jax import lax
from jax.experimental import pallas as pl
from jax.experimental.pallas import tpu as pltpu
from jax.experimental.pallas import tpu_sc as plsc

_TILE = 16
_HALF = 128
_DIM = 256
_NC = 2
_NS = 16
_NW = _NC * _NS
_LANES = 16

_CHUNK = 64
_GROWS = 2 * _CHUNK


def _pe_body(x_hbm, c1_hbm, c2_hbm, emb_hbm, out_hbm, stage_hbm,
             cbuf, acc2, gbuf, c1b, c2b, idx0, idx1, xb0, xb1, gb0, gb1,
             semx0, semx1, semg0, semg1, semo0, semo1):
    idxbuf = (idx0, idx1)
    xbuf = (xb0, xb1)
    gatbuf = (gb0, gb1)
    semx = (semx0, semx1)
    semg = (semg0, semg1)
    semo = (semo0, semo1)
    seq = x_hbm.shape[0]
    rows_per_w = seq // _NW
    nchunk = rows_per_w // _CHUNK
    min_per_sub = seq // _NS

    s = lax.axis_index("s")
    c = lax.axis_index("c")
    wid = s * _NC + c
    wbase = wid * rows_per_w

    mbase = s * min_per_sub
    for slot, src in ((0, c1_hbm), (1, c2_hbm)):
        pltpu.sync_copy(src.at[pl.ds(mbase, min_per_sub)], cbuf)
        acc2[pl.ds(slot * _LANES, _LANES)] = cbuf[pl.ds(0, _LANES)]

        def _minstep(i, _, _slot=slot):
            acc2[pl.ds(_slot * _LANES, _LANES)] = jnp.minimum(
                acc2[pl.ds(_slot * _LANES, _LANES)],
                cbuf[pl.ds(i * _LANES, _LANES)])
            return 0

        lax.fori_loop(1, min_per_sub // _LANES, _minstep, 0)

    pltpu.sync_copy(acc2, stage_hbm.at[c * _NS + s])
    plsc.subcore_barrier()
    pltpu.sync_copy(stage_hbm.at[pl.ds(c * _NS, _NS)], gbuf)

    m1 = gbuf[0, pl.ds(0, _LANES)]
    m2 = gbuf[0, pl.ds(_LANES, _LANES)]
    for t in range(1, _NS):
        m1 = jnp.minimum(m1, gbuf[t, pl.ds(0, _LANES)])
        m2 = jnp.minimum(m2, gbuf[t, pl.ds(_LANES, _LANES)])
    mx = m1[0]
    my = m2[0]
    for t in range(1, _LANES):
        mx = jnp.minimum(mx, m1[t])
        my = jnp.minimum(my, m2[t])

    pltpu.sync_copy(c1_hbm.at[pl.ds(wbase, rows_per_w)], c1b)
    pltpu.sync_copy(c2_hbm.at[pl.ds(wbase, rows_per_w)], c2b)

    mxv = jnp.full((_LANES,), mx, jnp.int32)
    myv = jnp.full((_LANES,), my, jnp.int32)
    tilev = jnp.full((_LANES,), _TILE, jnp.int32)
    offv = jnp.full((_LANES,), emb_hbm.shape[0] // 2, jnp.int32)
    groups_per_chunk = _CHUNK // _LANES

    def _build_idx(ci, slot):
        for k in range(groups_per_chunk):
            j = ci * groups_per_chunk + k
            vx = lax.div(c1b[pl.ds(j * _LANES, _LANES)] - mxv, tilev)
            idxbuf[slot][pl.ds(k * _LANES, _LANES)] = vx
            vy = lax.div(c2b[pl.ds(j * _LANES, _LANES)] - myv, tilev) + offv
            idxbuf[slot][pl.ds(_CHUNK + k * _LANES, _LANES)] = vy

    def _start(ci, slot):
        row0 = wbase + ci * _CHUNK
        xcp = pltpu.async_copy(x_hbm.at[pl.ds(row0, _CHUNK)],
                               xbuf[slot], semx[slot])
        gcp = pltpu.async_copy(emb_hbm.at[idxbuf[slot]],
                               gatbuf[slot], semg[slot])
        return xcp, gcp

    _build_idx(0, 0)
    inflight = {0: _start(0, 0)}
    ocp = {}
    for ci in range(nchunk):
        slot = ci % 2
        nxt = 1 - slot
        if ci + 1 < nchunk:
            _build_idx(ci + 1, nxt)
            if ci - 1 >= 0:
                ocp.pop(nxt).wait()
            inflight[nxt] = _start(ci + 1, nxt)
        xcp, gcp = inflight.pop(slot)
        xcp.wait()
        gcp.wait()

        xb = xbuf[slot]
        gb = gatbuf[slot]

        def _addrow(r, _):
            for h in range(_HALF // _LANES):
                sl = pl.ds(h * _LANES, _LANES)
                sr = pl.ds(_HALF + h * _LANES, _LANES)
                xb[r, sl] = xb[r, sl] + gb[r, sl]
                xb[r, sr] = xb[r, sr] + gb[_CHUNK + r, sl]
            return 0

        lax.fori_loop(0, _CHUNK, _addrow, 0)
        row0 = wbase + ci * _CHUNK
        ocp[slot] = pltpu.async_copy(xb, out_hbm.at[pl.ds(row0, _CHUNK)],
                                     semo[slot])
    for slot in list(ocp):
        ocp.pop(slot).wait()


def kernel(x, coords, emb_x, emb_y):
    seq, dim = x.shape
    c1 = coords[:, 1].astype(jnp.int32)
    c2 = coords[:, 2].astype(jnp.int32)
    emb_cat = jnp.concatenate([emb_x, emb_y], axis=0)

    rows_per_w = seq // _NW
    min_per_sub = seq // _NS

    run = pl.kernel(
        _pe_body,
        out_type=(
            jax.ShapeDtypeStruct((seq, dim), jnp.float32),
            jax.ShapeDtypeStruct((_NW, 2 * _LANES), jnp.int32),
        ),
        mesh=plsc.VectorSubcoreMesh(core_axis_name="c", subcore_axis_name="s"),
        scratch_types=[
            pltpu.VMEM((min_per_sub,), jnp.int32),
            pltpu.VMEM((2 * _LANES,), jnp.int32),
            pltpu.VMEM((_NS, 2 * _LANES), jnp.int32),
            pltpu.VMEM((rows_per_w,), jnp.int32),
            pltpu.VMEM((rows_per_w,), jnp.int32),
            pltpu.VMEM((_GROWS,), jnp.int32),
            pltpu.VMEM((_GROWS,), jnp.int32),
            pltpu.VMEM((_CHUNK, _DIM), jnp.float32),
            pltpu.VMEM((_CHUNK, _DIM), jnp.float32),
            pltpu.VMEM((_GROWS, _HALF), jnp.float32),
            pltpu.VMEM((_GROWS, _HALF), jnp.float32),
            pltpu.SemaphoreType.DMA,
            pltpu.SemaphoreType.DMA,
            pltpu.SemaphoreType.DMA,
            pltpu.SemaphoreType.DMA,
            pltpu.SemaphoreType.DMA,
            pltpu.SemaphoreType.DMA,
        ],
    )
    out, _ = run(x, c1, c2, emb_cat)
    return out
